# Initial kernel scaffold; baseline (speedup 1.0000x reference)
import jax
import jax.numpy as jnp
from jax.experimental import pallas as pl


def kernel(x, edge_index, edge_weight, W1, b1, W2, b2):
    raise NotImplementedError("write your pallas kernel here")


# speedup vs baseline: 9.3757x; 9.3757x over previous
import functools

import jax
import jax.numpy as jnp
from jax import lax
from jax.experimental import pallas as pl
from jax.experimental.pallas import tpu as pltpu
from jax.experimental.pallas import tpu_sc as plsc

N = 10000
E = 320000
D = 128

NC = 2
NS = 16
NW = NC * NS
NPAD = 10112
RPT = NPAD // NS
EPW = E // NW
C = 80
CHUNKS = EPW // C
CD = 2000
CHUNKS_D = EPW // CD


@functools.cache
def _sc_kernels():
    mesh = plsc.VectorSubcoreMesh(core_axis_name="c", subcore_axis_name="s",
                                  num_cores=NC, num_subcores=NS)

    @functools.partial(
        pl.kernel,
        out_type=jax.ShapeDtypeStruct((NC * NPAD,), jnp.float32),
        mesh=mesh,
        scratch_types=[
            pltpu.VMEM((CD,), jnp.int32),
            pltpu.VMEM((CD,), jnp.float32),
            pltpu.VMEM((640,), jnp.float32),
            pltpu.VMEM_SHARED((NPAD,), jnp.float32),
        ],
    )
    def deg_pass(row_hbm, w_hbm, parts_hbm, row_v, w_v, stage_v, deg_sh):
        c = lax.axis_index("c")
        s = lax.axis_index("s")
        wid = c * NS + s
        for i in range(640 // 16):
            stage_v[pl.ds(i * 16, 16)] = jnp.zeros((16,), jnp.float32)
        pltpu.sync_copy(stage_v.at[pl.ds(0, RPT)],
                        deg_sh.at[pl.ds(s * RPT, RPT)])
        plsc.subcore_barrier()

        def chunk(ch, carry):
            base = wid * EPW + ch * CD
            pltpu.sync_copy(row_hbm.at[pl.ds(base, CD)], row_v)
            pltpu.sync_copy(w_hbm.at[pl.ds(base, CD)], w_v)
            pltpu.sync_copy(w_v, deg_sh.at[row_v], add=True)
            return carry

        lax.fori_loop(0, CHUNKS_D, chunk, 0)
        plsc.subcore_barrier()
        pltpu.sync_copy(deg_sh.at[pl.ds(s * RPT, RPT)],
                        stage_v.at[pl.ds(0, RPT)])
        pltpu.sync_copy(stage_v.at[pl.ds(0, RPT)],
                        parts_hbm.at[pl.ds(c * NPAD + s * RPT, RPT)])

    @functools.partial(
        pl.kernel,
        out_type=jax.ShapeDtypeStruct((NC * NPAD, D), jnp.float32),
        mesh=mesh,
        scratch_types=[
            pltpu.VMEM((C,), jnp.int32),
            pltpu.VMEM((C,), jnp.int32),
            pltpu.VMEM((C,), jnp.float32),
            pltpu.VMEM((C, D), jnp.float32),
            pltpu.VMEM_SHARED((NPAD, D), jnp.float32),
            pltpu.SemaphoreType.DMA,
        ],
    )
    def msg_pass(y_hbm, row_hbm, col_hbm, w_hbm, parts_hbm,
                 row_v, col_v, w_v, rows_v, out_sh, sem):
        c = lax.axis_index("c")
        s = lax.axis_index("s")
        wid = c * NS + s

        def zero_row(i, carry):
            for k in range(8):
                rows_v[i, pl.ds(k * 16, 16)] = jnp.zeros((16,), jnp.float32)
            return carry

        lax.fori_loop(0, C, zero_row, 0)
        for off in range(0, RPT, C):
            ln = min(C, RPT - off)
            pltpu.sync_copy(rows_v.at[pl.ds(0, ln)],
                            out_sh.at[pl.ds(s * RPT + off, ln)])
        plsc.subcore_barrier()

        def chunk(ch, carry):
            base = wid * EPW + ch * C
            pltpu.sync_copy(row_hbm.at[pl.ds(base, C)], row_v)
            pltpu.sync_copy(col_hbm.at[pl.ds(base, C)], col_v)
            pltpu.sync_copy(w_hbm.at[pl.ds(base, C)], w_v)
            pltpu.async_copy(y_hbm.at[row_v], rows_v, sem).wait()

            def group(g, gcarry):
                w16 = w_v[pl.ds(g * 16, 16)]
                for j in range(16):
                    e = g * 16 + j
                    wj = jnp.full((16,), w16[j], jnp.float32)
                    for k in range(8):
                        sl = pl.ds(k * 16, 16)
                        rows_v[e, sl] = rows_v[e, sl] * wj
                return gcarry

            lax.fori_loop(0, C // 16, group, 0)
            pltpu.sync_copy(rows_v, out_sh.at[col_v], add=True)
            return carry

        lax.fori_loop(0, CHUNKS, chunk, 0)
        plsc.subcore_barrier()
        for off in range(0, RPT, C):
            ln = min(C, RPT - off)
            pltpu.sync_copy(out_sh.at[pl.ds(s * RPT + off, ln)],
                            rows_v.at[pl.ds(0, ln)])
            pltpu.sync_copy(rows_v.at[pl.ds(0, ln)],
                            parts_hbm.at[pl.ds(c * NPAD + s * RPT + off, ln)])

    return deg_pass, msg_pass


def _mm_body(x_ref, wt_ref, b_ref, o_ref):
    o_ref[...] = jnp.dot(x_ref[...], wt_ref[...],
                         preferred_element_type=jnp.float32) + b_ref[...]


_mm = pl.pallas_call(
    _mm_body, out_shape=jax.ShapeDtypeStruct((NPAD, D), jnp.float32))


def _scale_body(dp_ref, z_ref, y_ref, dis_ref):
    deg = dp_ref[0] + dp_ref[1]
    dis = jnp.where(deg > 0, lax.rsqrt(deg), 0.0)
    dis_ref[...] = dis
    y_ref[...] = dis * z_ref[...]


_scale = pl.pallas_call(
    _scale_body,
    out_shape=[jax.ShapeDtypeStruct((NPAD, D), jnp.float32),
               jax.ShapeDtypeStruct((NPAD, 1), jnp.float32)])


def _layer2_body(p_ref, dis_ref, wt_ref, b_ref, y_ref):
    dis = dis_ref[...]
    h = jnp.maximum(dis * (p_ref[0] + p_ref[1]), 0.0)
    y_ref[...] = dis * (jnp.dot(h, wt_ref[...],
                                preferred_element_type=jnp.float32) + b_ref[...])


_layer2 = pl.pallas_call(
    _layer2_body, out_shape=jax.ShapeDtypeStruct((NPAD, D), jnp.float32))


def _final_body(q_ref, dis_ref, o_ref):
    o_ref[...] = dis_ref[...] * (q_ref[0] + q_ref[1])


_final = pl.pallas_call(
    _final_body, out_shape=jax.ShapeDtypeStruct((NPAD, D), jnp.float32))


def kernel(x, edge_index, edge_weight, W1, b1, W2, b2):
    deg_pass, msg_pass = _sc_kernels()
    row = edge_index[0]
    col = edge_index[1]
    xp = jnp.pad(x, ((0, NPAD - N), (0, 0)))

    deg_parts = deg_pass(row, edge_weight)
    z1 = _mm(xp, W1.T, b1.reshape(1, D))
    y1, dis = _scale(deg_parts.reshape(NC, NPAD, 1), z1)
    p = msg_pass(y1, row, col, edge_weight).reshape(NC, NPAD, D)
    y2 = _layer2(p, dis, W2.T, b2.reshape(1, D))
    q = msg_pass(y2, row, col, edge_weight).reshape(NC, NPAD, D)
    out = _final(q, dis)
    return out[:N]
